# baseline (speedup 1.0000x reference)
import functools

import jax
import jax.numpy as jnp
from jax.experimental import pallas as pl
from jax.experimental.pallas import tpu as pltpu

Q = 4096
K = 100000
D = 128
TOPK = 32

BQ = 256
BK = 2048
KPAD = ((K + BK - 1) // BK) * BK


def _score_body(q_ref, k_ref, o_ref):
    q = q_ref[...]
    k = k_ref[...]
    q_sq = jnp.sum(q * q, axis=1, keepdims=True)
    k_sq = jnp.sum(k * k, axis=1)[None, :]
    dots = jnp.dot(q, k.T, preferred_element_type=jnp.float32)
    o_ref[...] = -(q_sq - 2.0 * dots + k_sq)


@jax.jit
def _scores(queries, keys_padded):
    grid = (Q // BQ, KPAD // BK)
    return pl.pallas_call(
        _score_body,
        grid=grid,
        in_specs=[
            pl.BlockSpec((BQ, D), lambda i, j: (i, 0)),
            pl.BlockSpec((BK, D), lambda i, j: (j, 0)),
        ],
        out_specs=pl.BlockSpec((BQ, BK), lambda i, j: (i, j)),
        out_shape=jax.ShapeDtypeStruct((Q, KPAD), jnp.float32),
    )(queries, keys_padded)


def kernel(queries, keys, k):
    keys_padded = jnp.pad(keys, ((0, KPAD - K), (0, 0)))
    scores = _scores(queries, keys_padded)
    vals, idx = jax.lax.top_k(scores[:, :K], TOPK)
    k_zero = jnp.asarray(k) * 0
    vals = vals + k_zero.astype(vals.dtype)
    idx = idx + k_zero.astype(idx.dtype)
    return vals, idx


# speedup vs baseline: 5.0038x; 5.0038x over previous
import jax
import jax.numpy as jnp
from jax.experimental import pallas as pl
from jax.experimental.pallas import tpu as pltpu

Q = 4096
K = 100000
D = 128
TOPK = 32

BQ = 32
G = 128
NCH = 8
KPAD = 100352
CHK = KPAD // NCH
NG = KPAD // G
GCH = NG // NCH
NGP = NCH * G
CAP = 64

NEG = -1e30


def _body(q_ref, kt_ref, vals_ref, pos_ref, cgrp_ref, s3_ref, gm_ref, sc2_ref):
    j = pl.program_id(1)

    q = q_ref[...]
    kt = kt_ref[...]
    q_sq = jnp.sum(q * q, axis=1, keepdims=True)
    k_sq = jnp.sum(kt * kt, axis=0, keepdims=True)
    dots = jnp.dot(q, kt, preferred_element_type=jnp.float32)
    score = -(q_sq - 2.0 * dots + k_sq)

    col = j * CHK + jax.lax.broadcasted_iota(jnp.int32, (BQ, CHK), 1)
    score = jnp.where(col >= K, NEG, score)

    score3 = score.reshape(BQ, GCH, G)
    score3 = jnp.pad(score3, ((0, 0), (0, G - GCH), (0, 0)),
                     constant_values=NEG)
    s3_ref[:, pl.ds(j * G, G), :] = score3
    gm_ref[:, pl.ds(j * G, G)] = jnp.max(score3, axis=2)

    @pl.when(j == NCH - 1)
    def _epilogue():
        gm = gm_ref[...]

        v = gm
        m = None
        for _ in range(TOPK):
            m = jnp.max(v, axis=1, keepdims=True)
            v = jnp.where(v == m, NEG, v)
        t = m

        mask = jnp.where(gm >= t, 1.0, 0.0)
        count = jnp.sum(mask, axis=1, keepdims=True)

        iu = jax.lax.broadcasted_iota(jnp.int32, (NGP, NGP), 0)
        ju = jax.lax.broadcasted_iota(jnp.int32, (NGP, NGP), 1)
        upper = jnp.where(iu < ju, 1.0, 0.0)
        rank = jnp.dot(mask, upper, preferred_element_type=jnp.float32)

        slot_iota = jax.lax.broadcasted_iota(jnp.int32, (BQ, NGP, CAP), 2)
        rank_i = rank.astype(jnp.int32)
        e3 = jnp.where(
            (rank_i[:, :, None] == slot_iota) & (mask[:, :, None] > 0.0),
            1.0, 0.0)

        iota_c = jax.lax.broadcasted_iota(
            jnp.int32, (NGP, 1), 0).astype(jnp.float32)
        for r in range(BQ):
            er = e3[r]
            sr = s3_ref[r]
            cand = jax.lax.dot_general(
                er, sr, (((0,), (0,)), ((), ())),
                preferred_element_type=jnp.float32,
                precision=jax.lax.Precision.HIGHEST)
            sc2_ref[r] = cand
            cg = jnp.sum(er * iota_c, axis=0)
            cgrp_ref[pl.ds(r, 1), :] = cg[None, :].astype(jnp.int32)

        fc = sc2_ref[...]
        slot3 = jax.lax.broadcasted_iota(jnp.int32, (BQ, CAP, G), 1)
        count_i = count.astype(jnp.int32)
        fc = fc + jnp.where(slot3 >= count_i[:, :, None], NEG, 0.0)

        pos3 = (G * jax.lax.broadcasted_iota(jnp.int32, (BQ, CAP, G), 1)
                + jax.lax.broadcasted_iota(jnp.int32, (BQ, CAP, G), 2))
        big = jnp.int32(2**30)
        for r in range(TOPK):
            mv = jnp.max(fc, axis=(1, 2))
            eq = fc == mv[:, None, None]
            pick = jnp.min(jnp.where(eq, pos3, big), axis=(1, 2))
            vals_ref[:, pl.ds(r, 1)] = mv[:, None]
            pos_ref[:, pl.ds(r, 1)] = pick[:, None]
            fc = jnp.where(pos3 == pick[:, None, None], NEG, fc)


@jax.jit
def _topk_kernel(queries, keys_t):
    grid = (Q // BQ, NCH)
    return pl.pallas_call(
        _body,
        grid=grid,
        in_specs=[
            pl.BlockSpec((BQ, D), lambda i, j: (i, 0)),
            pl.BlockSpec((D, CHK), lambda i, j: (0, j)),
        ],
        out_specs=[
            pl.BlockSpec((BQ, TOPK), lambda i, j: (i, 0)),
            pl.BlockSpec((BQ, TOPK), lambda i, j: (i, 0)),
            pl.BlockSpec((BQ, CAP), lambda i, j: (i, 0)),
        ],
        out_shape=[
            jax.ShapeDtypeStruct((Q, TOPK), jnp.float32),
            jax.ShapeDtypeStruct((Q, TOPK), jnp.int32),
            jax.ShapeDtypeStruct((Q, CAP), jnp.int32),
        ],
        scratch_shapes=[
            pltpu.VMEM((BQ, NGP, G), jnp.float32),
            pltpu.VMEM((BQ, NGP), jnp.float32),
            pltpu.VMEM((BQ, CAP, G), jnp.float32),
        ],
        compiler_params=pltpu.CompilerParams(
            dimension_semantics=("parallel", "arbitrary")),
    )(queries, keys_t)


def kernel(queries, keys, k):
    keys_t = jnp.pad(keys, ((0, KPAD - K), (0, 0))).T
    vals, pos, cgrp = _topk_kernel(queries, keys_t)
    s1 = pos // G
    j1 = pos % G
    g = jnp.take_along_axis(cgrp, s1, axis=1)
    g_real = (g // G) * GCH + (g % G)
    idx = g_real * G + j1
    return vals, idx
